# baseline (speedup 1.0000x reference)
import jax
import jax.numpy as jnp
from jax.experimental import pallas as pl
from jax.experimental.pallas import tpu as pltpu

BATCH = 4
SEQ_LEN = 8192
D_MODEL = 4096
N_EXPERTS = 64
EXPERT_CAPACITY = 160

BLK_S = 1024
BPB = SEQ_LEN // BLK_S
NBUF = 3


def _router_block(x_hbm, w_ref, b_ref, ei_ref, tp_ref, rp_ref, aux_ref,
                  buf_ref, sem, carry_ref, fi_ref, pi_ref):
    i = pl.program_id(0)
    n = pl.num_programs(0)

    def copy(j):
        slot = jax.lax.rem(j, NBUF)
        return pltpu.make_async_copy(
            x_hbm.at[pl.ds(j * BLK_S, BLK_S), :], buf_ref.at[slot],
            sem.at[slot])

    @pl.when(i == 0)
    def _prologue():
        copy(0).start()
        copy(1).start()

    @pl.when(i + 2 < n)
    def _prefetch():
        copy(i + 2).start()

    copy(i).wait()

    @pl.when(i % BPB == 0)
    def _reset():
        carry_ref[...] = jnp.zeros_like(carry_ref)
        fi_ref[...] = jnp.zeros_like(fi_ref)
        pi_ref[...] = jnp.zeros_like(pi_ref)

    x = buf_ref[jax.lax.rem(i, NBUF)]
    logits = jnp.dot(x, w_ref[...],
                     preferred_element_type=jnp.float32) + b_ref[...]
    m = jnp.max(logits, axis=-1, keepdims=True)
    e = jnp.exp(logits - m)
    s = jnp.sum(e, axis=-1, keepdims=True)
    inv = 1.0 / s
    probs = e * inv
    rp_ref[...] = probs

    maxp = inv
    tp_ref[...] = maxp

    eq = (probs == maxp).astype(jnp.float32)
    E = eq.shape[1]
    le_r = jax.lax.broadcasted_iota(jnp.int32, (E, E), 0)
    le_c = jax.lax.broadcasted_iota(jnp.int32, (E, E), 1)
    upper = (le_r < le_c).astype(jnp.float32)
    prior = jax.lax.dot_general(
        eq, upper, (((1,), (0,)), ((), ())),
        preferred_element_type=jnp.float32)
    onehot_f = eq * (prior == 0.0)

    G = BLK_S // 8
    grp = jnp.sum(onehot_f.reshape(G, 8, E), axis=1)
    rowg = jax.lax.broadcasted_iota(jnp.int32, (G, G), 0)
    colg = jax.lax.broadcasted_iota(jnp.int32, (G, G), 1)
    tri_strict = (rowg > colg).astype(jnp.float32)
    excl = jax.lax.dot_general(
        tri_strict, grp, (((1,), (0,)), ((), ())),
        preferred_element_type=jnp.float32)
    seed = excl + carry_ref[...]
    seed_rows = jnp.pad(seed[:, None, :],
                        ((0, 0), (0, 7), (0, 0))).reshape(BLK_S, E)
    y = onehot_f + seed_rows
    rowmod = jax.lax.broadcasted_iota(jnp.int32, (BLK_S, 1), 0) % 8
    for k in (1, 2, 4):
        y = y + jnp.where(rowmod >= k, jnp.roll(y, k, axis=0), 0.0)
    prio = y
    keep = prio <= EXPERT_CAPACITY
    kept = jnp.where(keep, onehot_f, 0.0)
    ei_ref[...] = kept.astype(jnp.int32)

    carry_ref[...] = prio[BLK_S - 1:BLK_S, :]
    fi_ref[...] += jnp.sum(kept, axis=0, keepdims=True)
    pi_ref[...] += jnp.sum(probs, axis=0, keepdims=True)

    @pl.when(i % BPB == BPB - 1)
    def _aux():
        partial = (N_EXPERTS / (BATCH * float(SEQ_LEN) * float(SEQ_LEN))) * \
            jnp.sum(fi_ref[...] * pi_ref[...])

        @pl.when(i == BPB - 1)
        def _init():
            aux_ref[...] = jnp.full((1, 1), partial, jnp.float32)

        @pl.when(i != BPB - 1)
        def _acc():
            aux_ref[...] += partial


@jax.jit
def kernel(hidden_states, W, b):
    B, S, D = hidden_states.shape
    E = W.shape[1]
    TOK = B * S
    hs = hidden_states.reshape(TOK, D)
    grid = (TOK // BLK_S,)

    ei, tp, rp, aux = pl.pallas_call(
        _router_block,
        grid=grid,
        in_specs=[
            pl.BlockSpec(memory_space=pl.ANY),
            pl.BlockSpec((D, E), lambda i: (0, 0)),
            pl.BlockSpec((1, E), lambda i: (0, 0)),
        ],
        out_specs=[
            pl.BlockSpec((BLK_S, E), lambda i: (i, 0)),
            pl.BlockSpec((BLK_S, 1), lambda i: (i, 0)),
            pl.BlockSpec((BLK_S, E), lambda i: (i, 0)),
            pl.BlockSpec((1, 1), lambda i: (0, 0)),
        ],
        out_shape=[
            jax.ShapeDtypeStruct((TOK, E), jnp.int32),
            jax.ShapeDtypeStruct((TOK, 1), jnp.float32),
            jax.ShapeDtypeStruct((TOK, E), jnp.float32),
            jax.ShapeDtypeStruct((1, 1), jnp.float32),
        ],
        scratch_shapes=[
            pltpu.VMEM((NBUF, BLK_S, D), jnp.float32),
            pltpu.SemaphoreType.DMA((NBUF,)),
            pltpu.VMEM((1, E), jnp.float32),
            pltpu.VMEM((1, E), jnp.float32),
            pltpu.VMEM((1, E), jnp.float32),
        ],
        compiler_params=pltpu.CompilerParams(
            dimension_semantics=("arbitrary",)),
    )(hs, W, b.reshape(1, E))

    return (ei.reshape(B, S, E), tp.reshape(B, S, 1),
            rp.reshape(B, S, E), aux[0, 0])


# speedup vs baseline: 1.0185x; 1.0185x over previous
import jax
import jax.numpy as jnp
from jax.experimental import pallas as pl
from jax.experimental.pallas import tpu as pltpu

BATCH = 4
SEQ_LEN = 8192
D_MODEL = 4096
N_EXPERTS = 64
EXPERT_CAPACITY = 160

BLK_S = 1024
BPB = SEQ_LEN // BLK_S


def _router_block(xa_ref, xb_ref, w_ref, b_ref, ei_ref, tp_ref, rp_ref, aux_ref,
                  carry_ref, fi_ref, pi_ref):
    bb = pl.program_id(0)
    i = pl.program_id(1)

    @pl.when(i == 0)
    def _reset():
        carry_ref[...] = jnp.zeros_like(carry_ref)
        fi_ref[...] = jnp.zeros_like(fi_ref)
        pi_ref[...] = jnp.zeros_like(pi_ref)

    xa = xa_ref[0]
    xb = xb_ref[0]
    logits = (jnp.dot(xa, w_ref[0], preferred_element_type=jnp.float32) +
              jnp.dot(xb, w_ref[1], preferred_element_type=jnp.float32) +
              b_ref[...])
    m = jnp.max(logits, axis=-1, keepdims=True)
    e = jnp.exp(logits - m)
    s = jnp.sum(e, axis=-1, keepdims=True)
    inv = 1.0 / s
    probs = e * inv
    rp_ref[0] = probs

    maxp = inv
    tp_ref[0] = maxp

    eq = (probs == maxp).astype(jnp.float32)
    E = eq.shape[1]
    le_r = jax.lax.broadcasted_iota(jnp.int32, (E, E), 0)
    le_c = jax.lax.broadcasted_iota(jnp.int32, (E, E), 1)
    upper = (le_r < le_c).astype(jnp.float32)
    prior = jax.lax.dot_general(
        eq, upper, (((1,), (0,)), ((), ())),
        preferred_element_type=jnp.float32)
    onehot_f = eq * (prior == 0.0)

    G = BLK_S // 8
    grp = jnp.sum(onehot_f.reshape(G, 8, E), axis=1)
    rowg = jax.lax.broadcasted_iota(jnp.int32, (G, G), 0)
    colg = jax.lax.broadcasted_iota(jnp.int32, (G, G), 1)
    tri_strict = (rowg > colg).astype(jnp.float32)
    excl = jax.lax.dot_general(
        tri_strict, grp, (((1,), (0,)), ((), ())),
        preferred_element_type=jnp.float32)
    seed = excl + carry_ref[...]
    seed_rows = jnp.pad(seed[:, None, :],
                        ((0, 0), (0, 7), (0, 0))).reshape(BLK_S, E)
    y = onehot_f + seed_rows
    rowmod = jax.lax.broadcasted_iota(jnp.int32, (BLK_S, 1), 0) % 8
    for k in (1, 2, 4):
        y = y + jnp.where(rowmod >= k, jnp.roll(y, k, axis=0), 0.0)
    prio = y
    keep = prio <= EXPERT_CAPACITY
    kept = jnp.where(keep, onehot_f, 0.0)
    ei_ref[0] = kept.astype(jnp.int32)

    carry_ref[...] = prio[BLK_S - 1:BLK_S, :]
    fi_ref[...] += jnp.sum(kept, axis=0, keepdims=True)
    pi_ref[...] += jnp.sum(probs, axis=0, keepdims=True)

    @pl.when(i == BPB - 1)
    def _aux():
        partial = (N_EXPERTS / (BATCH * float(SEQ_LEN) * float(SEQ_LEN))) * \
            jnp.sum(fi_ref[...] * pi_ref[...])

        @pl.when(bb == 0)
        def _init():
            aux_ref[...] = jnp.full((1, 1), partial, jnp.float32)

        @pl.when(bb != 0)
        def _acc():
            aux_ref[...] += partial


@jax.jit
def kernel(hidden_states, W, b):
    B, S, D = hidden_states.shape
    E = W.shape[1]
    grid = (B, S // BLK_S)

    ei, tp, rp, aux = pl.pallas_call(
        _router_block,
        grid=grid,
        in_specs=[
            pl.BlockSpec((1, BLK_S, D // 2), lambda b_, i: (b_, i, 0)),
            pl.BlockSpec((1, BLK_S, D // 2), lambda b_, i: (b_, i, 1)),
            pl.BlockSpec((2, D // 2, E), lambda b_, i: (0, 0, 0)),
            pl.BlockSpec((1, E), lambda b_, i: (0, 0)),
        ],
        out_specs=[
            pl.BlockSpec((1, BLK_S, E), lambda b_, i: (b_, i, 0)),
            pl.BlockSpec((1, BLK_S, 1), lambda b_, i: (b_, i, 0)),
            pl.BlockSpec((1, BLK_S, E), lambda b_, i: (b_, i, 0)),
            pl.BlockSpec((1, 1), lambda b_, i: (0, 0)),
        ],
        out_shape=[
            jax.ShapeDtypeStruct((B, S, E), jnp.int32),
            jax.ShapeDtypeStruct((B, S, 1), jnp.float32),
            jax.ShapeDtypeStruct((B, S, E), jnp.float32),
            jax.ShapeDtypeStruct((1, 1), jnp.float32),
        ],
        scratch_shapes=[
            pltpu.VMEM((1, E), jnp.float32),
            pltpu.VMEM((1, E), jnp.float32),
            pltpu.VMEM((1, E), jnp.float32),
        ],
        compiler_params=pltpu.CompilerParams(
            dimension_semantics=("arbitrary", "arbitrary")),
    )(hidden_states, hidden_states, W.reshape(2, D // 2, E), b.reshape(1, E))

    return (ei, tp, rp, aux[0, 0])
